# baseline (speedup 1.0000x reference)
import functools
import math

import jax
import jax.numpy as jnp
from jax import lax
from jax.experimental import pallas as pl
from jax.experimental.pallas import tpu as pltpu
from jax.experimental.pallas import tpu_sc as plsc

_N = 2048
_SIZE = 32000
_CONF = 0.9
_FILL = 0.1 / (_SIZE - 2)
_C = _CONF * math.log(_CONF) + 0.1 * math.log(_FILL)

_SC_ROWS = 256


_BR = 128
_GR = (_N - _SC_ROWS) // _BR
_ROW_OFF = _SC_ROWS // _BR


def _tc_body(tgt_ref, x_ref, out_ref):
    i = pl.program_id(0)

    x = x_ref[...]
    tgt = tgt_ref[0]
    live = tgt != 0
    rowcoef = jnp.where(live, -_FILL, 0.0).astype(jnp.float32)

    dense = lax.dot_general(
        rowcoef, x,
        dimension_numbers=(((0,), (0,)), ((), ())),
        preferred_element_type=jnp.float32,
    )

    cnt = jnp.sum(live.astype(jnp.float32))
    partial = jnp.sum(dense) - dense[0, 0] + _C * cnt

    @pl.when(i == 0)
    def _init():
        out_ref[0, 0] = 0.0

    out_ref[0, 0] += partial


def _tc_call(tgt3, x):
    return pl.pallas_call(
        _tc_body,
        grid=(_GR,),
        in_specs=[
            pl.BlockSpec((1, _BR, 1), lambda i: (i + _ROW_OFF, 0, 0)),
            pl.BlockSpec((_BR, _SIZE), lambda i: (i + _ROW_OFF, 0)),
        ],
        out_specs=pl.BlockSpec(
            (1, 1), lambda i: (0, 0), memory_space=pltpu.SMEM
        ),
        out_shape=jax.ShapeDtypeStruct((1, 1), jnp.float32),
    )(tgt3, x)


_NC = 2
_NS = 16
_L = 16
_NW = _NC * _NS
_TPW = _N // _NW
_NB = _TPW // _L
_RPW = _SC_ROWS // _NW
_CW = 3200
_NCH = _SIZE // _CW

_sc_mesh = plsc.VectorSubcoreMesh(core_axis_name="c", subcore_axis_name="s")


@functools.partial(
    pl.kernel,
    mesh=_sc_mesh,
    out_type=jax.ShapeDtypeStruct((_NW * _L,), jnp.float32),
    scratch_types=[
        pltpu.VMEM((_TPW,), jnp.int32),
        pltpu.VMEM((_L,), jnp.int32),
        pltpu.VMEM((_L * 8, 128), jnp.float32),
        pltpu.VMEM((8, _CW), jnp.float32),
        pltpu.VMEM((8, _CW), jnp.float32),
        pltpu.VMEM((_L,), jnp.float32),
        pltpu.SemaphoreType.DMA,
        pltpu.SemaphoreType.DMA,
        pltpu.SemaphoreType.DMA,
    ],
    compiler_params=pltpu.CompilerParams(
        use_tc_tiling_on_sc=True, needs_layout_passes=False
    ),
)
def _sc_kernel(x_hbm, tgt_hbm, out_hbm, tgt_v, tgt8_v, gbuf, buf0, buf1,
               acc_v, gsem, sem0, sem1):
    wid = lax.axis_index("s") * _NC + lax.axis_index("c")

    rbase = wid * _RPW
    pltpu.sync_copy(tgt_hbm.at[pl.ds(rbase, _L)], tgt8_v)

    bufs = (buf0, buf1)
    sems = (sem0, sem1)

    def chunk_copy(c, b):
        return pltpu.make_async_copy(
            x_hbm.at[pl.ds(rbase, 8), pl.ds(c * _CW, _CW)],
            bufs[b], sems[b],
        )

    chunk_copy(0, 0).start()
    chunk_copy(1, 1).start()

    lane = lax.iota(jnp.int32, _L)
    low8 = lane < 8
    lane8 = jnp.bitwise_and(lane, 7)

    rowsums = [jnp.zeros((_L,), jnp.float32) for _ in range(8)]
    acc_0 = jnp.zeros((_L,), jnp.float32)
    for c in range(_NCH):
        b = c % 2
        chunk_copy(c, b).wait()
        buf = bufs[b]

        def body(l, carry):
            off = l * _L
            return tuple(
                carry[s] + buf[s, pl.ds(off, _L)] for s in range(8)
            )
        rowsums = list(lax.fori_loop(0, _CW // _L, body, tuple(rowsums)))

        if c == 0:
            t8 = plsc.load_gather(tgt8_v, [lane8])
            v0 = plsc.load_gather(buf, [lane8, jnp.zeros((_L,), jnp.int32)])
            acc_0 = jnp.where((t8 != 0) & low8, v0, 0.0)

        if c + 2 < _NCH:
            chunk_copy(c + 2, b).start()

    total = jnp.zeros((_L,), jnp.float32)
    for s in range(8):
        t_s = plsc.load_gather(tgt8_v, [jnp.full((_L,), s, jnp.int32)])
        total = total + jnp.where(t_s != 0, rowsums[s], 0.0)

    t8 = plsc.load_gather(tgt8_v, [lane8])
    cnt = jnp.where((t8 != 0) & low8, 1.0, 0.0)

    base = wid * _TPW
    pltpu.sync_copy(tgt_hbm.at[pl.ds(base, _TPW)], tgt_v)

    rowsel = lane * 8 + lane8

    acc_t = jnp.zeros((_L,), jnp.float32)
    for bt in range(_NB):
        t16 = tgt_v[pl.ds(bt * _L, _L)]
        for k in range(_L):
            r0 = pl.multiple_of(base + bt * _L + k - (k & 7), 8)
            c0 = pl.multiple_of((t16[k] >> 7) << 7, 128)
            pltpu.make_async_copy(
                x_hbm.at[pl.ds(r0, 8), pl.ds(c0, 128)],
                gbuf.at[pl.ds(k * 8, 8)], gsem,
            ).start()
        for k in range(_L):
            pltpu.make_async_copy(
                x_hbm.at[pl.ds(base, 8), pl.ds(0, 128)],
                gbuf.at[pl.ds(k * 8, 8)], gsem,
            ).wait()

        col = jnp.bitwise_and(t16, 127)
        vals = plsc.load_gather(gbuf, [rowsel, col])
        acc_t = acc_t + jnp.where(t16 != 0, vals, 0.0)

    acc_v[...] = (-_FILL * total + _FILL * acc_0 + _C * cnt
                  - (_CONF - _FILL) * acc_t)
    pltpu.sync_copy(acc_v, out_hbm.at[pl.ds(wid * _L, _L)])


def kernel(x, target):
    tgt3 = target[_SC_ROWS:].reshape(_GR, _BR, 1)
    dense = _tc_call(tgt3, x)
    sparse = _sc_kernel(x, target)
    return dense[0, 0] + jnp.sum(sparse)


# speedup vs baseline: 1.0078x; 1.0078x over previous
import functools
import math

import jax
import jax.numpy as jnp
from jax import lax
from jax.experimental import pallas as pl
from jax.experimental.pallas import tpu as pltpu
from jax.experimental.pallas import tpu_sc as plsc

_N = 2048
_SIZE = 32000
_CONF = 0.9
_FILL = 0.1 / (_SIZE - 2)
_C = _CONF * math.log(_CONF) + 0.1 * math.log(_FILL)


_BR = 128
_GR = _N // _BR
_HC = _SIZE // 2


def _tc_body(tgt_ref, xl_ref, xr_ref, out_ref):
    i = pl.program_id(0)

    tgt = tgt_ref[0]
    live = tgt != 0
    rowcoef = jnp.where(live, -_FILL, 0.0).astype(jnp.float32)

    dn = (((0,), (0,)), ((), ()))
    dense_l = lax.dot_general(rowcoef, xl_ref[...], dimension_numbers=dn,
                              preferred_element_type=jnp.float32)
    dense_r = lax.dot_general(rowcoef, xr_ref[...], dimension_numbers=dn,
                              preferred_element_type=jnp.float32)

    cnt = jnp.sum(live.astype(jnp.float32))
    partial = (jnp.sum(dense_l) + jnp.sum(dense_r) - dense_l[0, 0]
               + _C * cnt)

    @pl.when(i == 0)
    def _init():
        out_ref[0, 0] = 0.0

    out_ref[0, 0] += partial


def _tc_call(tgt3, x):
    return pl.pallas_call(
        _tc_body,
        grid=(_GR,),
        in_specs=[
            pl.BlockSpec((1, _BR, 1), lambda i: (i, 0, 0)),
            pl.BlockSpec((_BR, _HC), lambda i: (i, 0)),
            pl.BlockSpec((_BR, _HC), lambda i: (i, 1)),
        ],
        out_specs=pl.BlockSpec(
            (1, 1), lambda i: (0, 0), memory_space=pltpu.SMEM
        ),
        out_shape=jax.ShapeDtypeStruct((1, 1), jnp.float32),
    )(tgt3, x, x)


_NC = 2
_NS = 16
_L = 16
_NW = _NC * _NS
_TPW = _N // _NW
_NB = _TPW // _L

_sc_mesh = plsc.VectorSubcoreMesh(core_axis_name="c", subcore_axis_name="s")


@functools.partial(
    pl.kernel,
    mesh=_sc_mesh,
    out_type=jax.ShapeDtypeStruct((_NW * _L,), jnp.float32),
    scratch_types=[
        pltpu.VMEM((_TPW,), jnp.int32),
        pltpu.VMEM((_L * 8, 128), jnp.float32),
        pltpu.VMEM((_L,), jnp.float32),
        pltpu.SemaphoreType.DMA,
    ],
    compiler_params=pltpu.CompilerParams(
        use_tc_tiling_on_sc=True, needs_layout_passes=False
    ),
)
def _sc_kernel(x_hbm, tgt_hbm, out_hbm, tgt_v, buf, acc_v, sem):
    wid = lax.axis_index("s") * _NC + lax.axis_index("c")
    base = wid * _TPW
    pltpu.sync_copy(tgt_hbm.at[pl.ds(base, _TPW)], tgt_v)

    lane = lax.iota(jnp.int32, _L)
    rowsel = lane * 8 + jnp.bitwise_and(lane, 7)

    acc = jnp.zeros((_L,), jnp.float32)
    for b in range(_NB):
        t16 = tgt_v[pl.ds(b * _L, _L)]
        for k in range(_L):
            r0 = pl.multiple_of(base + b * _L + k - (k & 7), 8)
            c0 = pl.multiple_of((t16[k] >> 7) << 7, 128)
            pltpu.make_async_copy(
                x_hbm.at[pl.ds(r0, 8), pl.ds(c0, 128)],
                buf.at[pl.ds(k * 8, 8)], sem,
            ).start()
        for k in range(_L):
            pltpu.make_async_copy(
                x_hbm.at[pl.ds(base, 8), pl.ds(0, 128)],
                buf.at[pl.ds(k * 8, 8)], sem,
            ).wait()

        col = jnp.bitwise_and(t16, 127)
        vals = plsc.load_gather(buf, [rowsel, col])
        acc = acc + jnp.where(t16 != 0, vals, 0.0)

    acc_v[...] = -(_CONF - _FILL) * acc
    pltpu.sync_copy(acc_v, out_hbm.at[pl.ds(wid * _L, _L)])


def kernel(x, target):
    tgt3 = target.reshape(_GR, _BR, 1)
    dense = _tc_call(tgt3, x)
    sparse = _sc_kernel(x, target)
    return dense[0, 0] + jnp.sum(sparse)
